# Initial kernel scaffold; baseline (speedup 1.0000x reference)
import jax
import jax.numpy as jnp
from jax.experimental import pallas as pl


def kernel(x, edge_index, W1, b1, W2, b2, Wl, bl):
    raise NotImplementedError("write your pallas kernel here")


# speedup vs baseline: 15.6985x; 15.6985x over previous
import functools

import jax
import jax.numpy as jnp
from jax import lax
from jax.experimental import pallas as pl
from jax.experimental.pallas import tpu as pltpu
from jax.experimental.pallas import tpu_sc as plsc

N = 10000
E = 320000
DIN = 128
DH = 256
NC, NS = 2, 16
NW = NC * NS
EPT = 10240
K = 128
NV = 10016
NA = 10240
BLK = 2000

_mesh = plsc.VectorSubcoreMesh(core_axis_name="c", subcore_axis_name="s")


@functools.partial(
    pl.kernel,
    out_type=jax.ShapeDtypeStruct((NC, NA, 16), jnp.float32),
    mesh=_mesh,
    scratch_types=[
        pltpu.VMEM_SHARED((NA, 16), jnp.float32),
        pltpu.VMEM((1, K), jnp.int32),
        pltpu.VMEM((K, 16), jnp.float32),
        pltpu.VMEM((64, 16), jnp.float32),
    ],
)
def _hist_kernel(edges_hbm, out_hbm, hist_sp, idx_v, ones_v, zbuf_v):
    c = lax.axis_index("c")
    s = lax.axis_index("s")

    @pl.loop(0, 64)
    def _(r):
        zbuf_v[r] = jnp.zeros((16,), jnp.float32)

    @pl.loop(0, K)
    def _(r):
        ones_v[r] = jnp.full((16,), 1.0, jnp.float32)

    @pl.loop(0, 10)
    def _(j):
        pltpu.sync_copy(zbuf_v, hist_sp.at[pl.ds(s * 640 + j * 64, 64)])

    plsc.subcore_barrier()

    base = (c * NS + s) * EPT

    @pl.loop(0, EPT // K)
    def _(j):
        pltpu.sync_copy(edges_hbm.at[0, 1, pl.ds(base + j * K, K)], idx_v.at[0])
        pltpu.sync_copy(ones_v, hist_sp.at[idx_v.at[0]], add=True)

    plsc.subcore_barrier()
    pltpu.sync_copy(
        hist_sp.at[pl.ds(s * 640, 640)], out_hbm.at[c, pl.ds(s * 640, 640)]
    )


def _make_agg(table_rows, split_edges):
    nch = (EPT if split_edges else 2 * EPT) // K

    @functools.partial(
        pl.kernel,
        out_type=jax.ShapeDtypeStruct((NC, NA, DIN), jnp.float32),
        mesh=_mesh,
        scratch_types=[
            pltpu.VMEM_SHARED((NA, DIN), jnp.float32),
            pltpu.VMEM((2, K), jnp.int32),
            pltpu.VMEM((K, DIN), jnp.float32),
            pltpu.VMEM((64, DIN), jnp.float32),
            pltpu.SemaphoreType.DMA,
        ],
    )
    def agg(table_hbm, edges_hbm, out_hbm, acc_sp, eidx_v, rows_v, zbuf_v, sem):
        c = lax.axis_index("c")
        s = lax.axis_index("s")

        @pl.loop(0, 64)
        def _(r):
            for kk in range(DIN // 16):
                zbuf_v[r, pl.ds(kk * 16, 16)] = jnp.zeros((16,), jnp.float32)

        @pl.loop(0, 10)
        def _(j):
            pltpu.sync_copy(zbuf_v, acc_sp.at[pl.ds(s * 640 + j * 64, 64)])

        plsc.subcore_barrier()

        if split_edges:
            ebase = (c * NS + s) * EPT
            vrow = 0
        else:
            ebase = s * (2 * EPT)
            vrow = c

        @pl.loop(0, nch)
        def _(j):
            base = ebase + j * K
            pltpu.sync_copy(edges_hbm.at[vrow, :, pl.ds(base, K)], eidx_v)
            pltpu.async_copy(table_hbm.at[eidx_v.at[0]], rows_v, sem).wait()
            pltpu.sync_copy(rows_v, acc_sp.at[eidx_v.at[1]], add=True)

        plsc.subcore_barrier()
        pltpu.sync_copy(
            acc_sp.at[pl.ds(s * 640, 640)], out_hbm.at[c, pl.ds(s * 640, 640)]
        )

    return agg


_agg_l1 = _make_agg(NV, split_edges=True)
_agg_l2 = _make_agg(2 * NV, split_edges=False)


def _dis_from_hist(hist_blk):
    deg = hist_blk[0, :, 0:1] + hist_blk[1, :, 0:1] + 1.0
    return lax.rsqrt(deg)


def _scale_body(hist_ref, x_ref, xs_ref):
    dis = _dis_from_hist(hist_ref[...])
    xs_ref[...] = x_ref[...] * dis


_scale_call = pl.pallas_call(
    _scale_body,
    grid=(N // BLK,),
    in_specs=[
        pl.BlockSpec((NC, BLK, 16), lambda i: (0, i, 0)),
        pl.BlockSpec((BLK, DIN), lambda i: (i, 0)),
    ],
    out_specs=pl.BlockSpec((BLK, DIN), lambda i: (i, 0)),
    out_shape=jax.ShapeDtypeStruct((NV, DIN), jnp.float32),
)


def _l1_body(part_ref, xs1_ref, hist_ref, w1_ref, b1_ref, xs2_ref):
    dis = _dis_from_hist(hist_ref[...])
    z = (part_ref[0] + part_ref[1] + xs1_ref[...]) * dis
    h = jnp.dot(
        z, w1_ref[...],
        precision=lax.Precision.HIGHEST,
        preferred_element_type=jnp.float32,
    )
    h = jnp.maximum(h + b1_ref[...], 0.0) * dis
    xs2_ref[0] = h[:, :DIN]
    xs2_ref[1] = h[:, DIN:]


_l1_call = pl.pallas_call(
    _l1_body,
    grid=(N // BLK,),
    in_specs=[
        pl.BlockSpec((NC, BLK, DIN), lambda i: (0, i, 0)),
        pl.BlockSpec((BLK, DIN), lambda i: (i, 0)),
        pl.BlockSpec((NC, BLK, 16), lambda i: (0, i, 0)),
        pl.BlockSpec((DIN, DH), lambda i: (0, 0)),
        pl.BlockSpec((1, DH), lambda i: (0, 0)),
    ],
    out_specs=pl.BlockSpec((2, BLK, DIN), lambda i: (0, i, 0)),
    out_shape=jax.ShapeDtypeStruct((2, NV, DIN), jnp.float32),
)


def _l2_body(acc_ref, xs2_ref, hist_ref, w2_ref, b2_ref, wl_ref, bl_ref,
             out_ref, csum_ref):
    i = pl.program_id(0)
    dis = _dis_from_hist(hist_ref[...])
    z = (acc_ref[...] + xs2_ref[...]) * dis[None]
    zf = jnp.concatenate([z[0], z[1]], axis=1)
    h = jnp.dot(
        zf, w2_ref[...],
        precision=lax.Precision.HIGHEST,
        preferred_element_type=jnp.float32,
    )
    h = jnp.maximum(h + b2_ref[...], 0.0)
    ps = jnp.sum(h, axis=0, keepdims=True)

    @pl.when(i == 0)
    def _():
        csum_ref[...] = ps

    @pl.when(i != 0)
    def _():
        csum_ref[...] = csum_ref[...] + ps

    @pl.when(i == N // BLK - 1)
    def _():
        m = csum_ref[...] * (1.0 / N)
        out_ref[...] = jnp.dot(
            m, wl_ref[...],
            precision=lax.Precision.HIGHEST,
            preferred_element_type=jnp.float32,
        ) + bl_ref[...]


def _l2_call(acc2, xs2, hist, W2, b2, Wl, bl):
    nout = Wl.shape[1]
    return pl.pallas_call(
        _l2_body,
        grid=(N // BLK,),
        in_specs=[
            pl.BlockSpec((NC, BLK, DIN), lambda i: (0, i, 0)),
            pl.BlockSpec((NC, BLK, DIN), lambda i: (0, i, 0)),
            pl.BlockSpec((NC, BLK, 16), lambda i: (0, i, 0)),
            pl.BlockSpec((DH, DH), lambda i: (0, 0)),
            pl.BlockSpec((1, DH), lambda i: (0, 0)),
            pl.BlockSpec((DH, nout), lambda i: (0, 0)),
            pl.BlockSpec((1, nout), lambda i: (0, 0)),
        ],
        out_specs=pl.BlockSpec((1, nout), lambda i: (0, 0)),
        out_shape=jax.ShapeDtypeStruct((1, nout), jnp.float32),
        scratch_shapes=[pltpu.VMEM((1, DH), jnp.float32)],
    )(acc2, xs2, hist, W2, b2, Wl, bl)


def kernel(x, edge_index, W1, b1, W2, b2, Wl, bl):
    src = edge_index[0].astype(jnp.int32)
    dst = edge_index[1].astype(jnp.int32)

    npad = EPT - E // NW
    padv = N + (jnp.arange(npad, dtype=jnp.int32) % 16)
    padm = jnp.broadcast_to(padv, (NW, npad))
    srcp = jnp.concatenate([src.reshape(NW, E // NW), padm], 1).reshape(-1)
    dstp = jnp.concatenate([dst.reshape(NW, E // NW), padm], 1).reshape(-1)
    edges = jnp.stack(
        [jnp.stack([srcp, dstp]), jnp.stack([srcp + NV, dstp])]
    )

    hist = _hist_kernel(edges)
    xs1 = _scale_call(hist, x)
    part1 = _agg_l1(xs1, edges)
    xs2 = _l1_call(part1, xs1, hist, W1, b1.reshape(1, DH))
    acc2 = _agg_l2(xs2.reshape(2 * NV, DIN), edges)
    out = _l2_call(acc2, xs2, hist, W2,
                   b2.reshape(1, DH), Wl, bl.reshape(1, -1))
    return out.reshape(-1)
